# Initial kernel scaffold; baseline (speedup 1.0000x reference)
import jax
import jax.numpy as jnp
from jax.experimental import pallas as pl


def kernel(x, emb, W1, b1, W2, b2):
    raise NotImplementedError("write your pallas kernel here")


# speedup vs baseline: 2.1125x; 2.1125x over previous
import functools

import jax
import jax.numpy as jnp
from jax import lax
from jax.experimental import pallas as pl
from jax.experimental.pallas import tpu as pltpu
from jax.experimental.pallas import tpu_sc as plsc

VOCAB = 1000000
D = 64
HIDDEN = 84
OUT_D = 64
BATCH = 16384
HIST = 200

NC = 2
NS = 16
NW = NC * NS
S_PER_W = BATCH // NW
SB = 8
NA = 104
NB = HIST - NA


def _pool_sc(x, emb):
    x1 = x.reshape(BATCH * HIST)
    mesh = plsc.VectorSubcoreMesh(core_axis_name="c", subcore_axis_name="s")

    @functools.partial(
        pl.kernel,
        mesh=mesh,
        compiler_params=pltpu.CompilerParams(use_tc_tiling_on_sc=False),
        out_type=jax.ShapeDtypeStruct((BATCH * D,), jnp.float32),
        scratch_types=[
            pltpu.VMEM((SB * HIST,), jnp.int32),
            pltpu.VMEM((NA, D), jnp.float32),
            pltpu.VMEM((NB, D), jnp.float32),
            pltpu.VMEM((SB * D,), jnp.float32),
            pltpu.SemaphoreType.DMA,
        ],
    )
    def pool(x_hbm, emb_hbm, out_hbm, idx_v, rows_a, rows_b, ostage_v, sem):
        wid = lax.axis_index("s") * NC + lax.axis_index("c")
        base = wid * S_PER_W
        inv = jnp.float32(1.0 / HIST)

        def blk_body(blk, carry):
            sbase = base + blk * SB
            pltpu.sync_copy(x_hbm.at[pl.ds(sbase * HIST, SB * HIST)], idx_v)

            def samp_body(s, carry2):
                cp1 = pltpu.async_copy(
                    emb_hbm.at[idx_v.at[pl.ds(s * HIST, NA)]], rows_a, sem)
                cp2 = pltpu.async_copy(
                    emb_hbm.at[idx_v.at[pl.ds(s * HIST + NA, NB)]], rows_b, sem)
                cp1.wait()
                cp2.wait()

                def acc_body(j, accs):
                    a0, a1, a2, a3 = accs
                    a0 = a0 + rows_a[j, pl.ds(0, 16)] + rows_b[j, pl.ds(0, 16)]
                    a1 = a1 + rows_a[j, pl.ds(16, 16)] + rows_b[j, pl.ds(16, 16)]
                    a2 = a2 + rows_a[j, pl.ds(32, 16)] + rows_b[j, pl.ds(32, 16)]
                    a3 = a3 + rows_a[j, pl.ds(48, 16)] + rows_b[j, pl.ds(48, 16)]
                    return (a0, a1, a2, a3)

                def tail_body(j, accs):
                    a0, a1, a2, a3 = accs
                    a0 = a0 + rows_a[j, pl.ds(0, 16)]
                    a1 = a1 + rows_a[j, pl.ds(16, 16)]
                    a2 = a2 + rows_a[j, pl.ds(32, 16)]
                    a3 = a3 + rows_a[j, pl.ds(48, 16)]
                    return (a0, a1, a2, a3)

                z = jnp.zeros((16,), jnp.float32)
                accs = lax.fori_loop(0, NB, acc_body, (z, z, z, z))
                a0, a1, a2, a3 = lax.fori_loop(NB, NA, tail_body, accs)
                ostage_v[pl.ds(s * D + 0, 16)] = a0 * inv
                ostage_v[pl.ds(s * D + 16, 16)] = a1 * inv
                ostage_v[pl.ds(s * D + 32, 16)] = a2 * inv
                ostage_v[pl.ds(s * D + 48, 16)] = a3 * inv
                return carry2

            lax.fori_loop(0, SB, samp_body, 0)
            pltpu.sync_copy(ostage_v, out_hbm.at[pl.ds(sbase * D, SB * D)])
            return carry

        lax.fori_loop(0, S_PER_W // SB, blk_body, 0)

    return pool(x1, emb).reshape(BATCH, D)


def _mlp_body(p_ref, w1_ref, b1_ref, w2_ref, b2_ref, o_ref):
    h = jnp.tanh(
        jnp.dot(p_ref[...], w1_ref[...], preferred_element_type=jnp.float32)
        + b1_ref[...])
    o_ref[...] = jnp.tanh(
        jnp.dot(h, w2_ref[...], preferred_element_type=jnp.float32)
        + b2_ref[...])


def _mlp_tc(pooled, W1, b1, W2, b2):
    MB = 2048
    return pl.pallas_call(
        _mlp_body,
        grid=(BATCH // MB,),
        in_specs=[
            pl.BlockSpec((MB, D), lambda i: (i, 0)),
            pl.BlockSpec((D, HIDDEN), lambda i: (0, 0)),
            pl.BlockSpec((1, HIDDEN), lambda i: (0, 0)),
            pl.BlockSpec((HIDDEN, OUT_D), lambda i: (0, 0)),
            pl.BlockSpec((1, OUT_D), lambda i: (0, 0)),
        ],
        out_specs=pl.BlockSpec((MB, OUT_D), lambda i: (i, 0)),
        out_shape=jax.ShapeDtypeStruct((BATCH, OUT_D), jnp.float32),
    )(pooled, W1.T, b1[None, :], W2.T, b2[None, :])


def kernel(x, emb, W1, b1, W2, b2):
    pooled = _pool_sc(x, emb)
    return _mlp_tc(pooled, W1, b1, W2, b2)


# speedup vs baseline: 2.7347x; 1.2945x over previous
import functools

import jax
import jax.numpy as jnp
from jax import lax
from jax.experimental import pallas as pl
from jax.experimental.pallas import tpu as pltpu
from jax.experimental.pallas import tpu_sc as plsc

VOCAB = 1000000
D = 64
HIDDEN = 84
OUT_D = 64
BATCH = 16384
HIST = 200

NC = 2
NS = 16
NW = NC * NS
S_PER_W = BATCH // NW
SB = 32
C = 40
T = HIST // C


def _pool_sc(x, emb):
    x1 = x.reshape(BATCH * HIST)
    mesh = plsc.VectorSubcoreMesh(core_axis_name="c", subcore_axis_name="s")

    @functools.partial(
        pl.kernel,
        mesh=mesh,
        compiler_params=pltpu.CompilerParams(use_tc_tiling_on_sc=False),
        out_type=jax.ShapeDtypeStruct((BATCH * D,), jnp.float32),
        scratch_types=[
            pltpu.VMEM((SB * HIST,), jnp.int32),
            pltpu.VMEM((C, D), jnp.float32),
            pltpu.VMEM((C, D), jnp.float32),
            pltpu.VMEM((SB * D,), jnp.float32),
            pltpu.SemaphoreType.DMA,
            pltpu.SemaphoreType.DMA,
        ],
    )
    def pool(x_hbm, emb_hbm, out_hbm, idx_v, acc0, acc1, ostage_v, sem0, sem1):
        wid = lax.axis_index("s") * NC + lax.axis_index("c")
        base = wid * S_PER_W
        inv = jnp.float32(1.0 / HIST)
        z = jnp.zeros((16,), jnp.float32)

        def zero(acc):
            def zbody(j, carry):
                acc[j, pl.ds(0, 16)] = z
                acc[j, pl.ds(16, 16)] = z
                acc[j, pl.ds(32, 16)] = z
                acc[j, pl.ds(48, 16)] = z
                return carry
            lax.fori_loop(0, C, zbody, 0)

        def fire(s, acc, sem):
            cps = []
            for t in range(T):
                cps.append(pltpu.async_copy(
                    emb_hbm.at[idx_v.at[pl.ds(s * HIST + t * C, C)]],
                    acc, sem, add=True))
            return cps

        def reduce(s, acc):
            def rbody(j, accs):
                a0, a1, a2, a3 = accs
                return (a0 + acc[j, pl.ds(0, 16)],
                        a1 + acc[j, pl.ds(16, 16)],
                        a2 + acc[j, pl.ds(32, 16)],
                        a3 + acc[j, pl.ds(48, 16)])
            a0, a1, a2, a3 = lax.fori_loop(0, C, rbody, (z, z, z, z))
            ostage_v[pl.ds(s * D + 0, 16)] = a0 * inv
            ostage_v[pl.ds(s * D + 16, 16)] = a1 * inv
            ostage_v[pl.ds(s * D + 32, 16)] = a2 * inv
            ostage_v[pl.ds(s * D + 48, 16)] = a3 * inv

        def blk_body(blk, carry):
            sbase = base + blk * SB
            pltpu.sync_copy(x_hbm.at[pl.ds(sbase * HIST, SB * HIST)], idx_v)

            def pair_body(p, carry2):
                s0 = 2 * p
                s1 = 2 * p + 1
                zero(acc0)
                cps0 = fire(s0, acc0, sem0)
                zero(acc1)
                cps1 = fire(s1, acc1, sem1)
                for cp in cps0:
                    cp.wait()
                reduce(s0, acc0)
                for cp in cps1:
                    cp.wait()
                reduce(s1, acc1)
                return carry2

            lax.fori_loop(0, SB // 2, pair_body, 0)
            pltpu.sync_copy(ostage_v, out_hbm.at[pl.ds(sbase * D, SB * D)])
            return carry

        lax.fori_loop(0, S_PER_W // SB, blk_body, 0)

    return pool(x1, emb).reshape(BATCH, D)


def _mlp_body(p_ref, w1_ref, b1_ref, w2_ref, b2_ref, o_ref):
    h = jnp.tanh(
        jnp.dot(p_ref[...], w1_ref[...], preferred_element_type=jnp.float32)
        + b1_ref[...])
    o_ref[...] = jnp.tanh(
        jnp.dot(h, w2_ref[...], preferred_element_type=jnp.float32)
        + b2_ref[...])


def _mlp_tc(pooled, W1, b1, W2, b2):
    MB = 2048
    return pl.pallas_call(
        _mlp_body,
        grid=(BATCH // MB,),
        in_specs=[
            pl.BlockSpec((MB, D), lambda i: (i, 0)),
            pl.BlockSpec((D, HIDDEN), lambda i: (0, 0)),
            pl.BlockSpec((1, HIDDEN), lambda i: (0, 0)),
            pl.BlockSpec((HIDDEN, OUT_D), lambda i: (0, 0)),
            pl.BlockSpec((1, OUT_D), lambda i: (0, 0)),
        ],
        out_specs=pl.BlockSpec((MB, OUT_D), lambda i: (i, 0)),
        out_shape=jax.ShapeDtypeStruct((BATCH, OUT_D), jnp.float32),
    )(pooled, W1.T, b1[None, :], W2.T, b2[None, :])


def kernel(x, emb, W1, b1, W2, b2):
    pooled = _pool_sc(x, emb)
    return _mlp_tc(pooled, W1, b1, W2, b2)


# speedup vs baseline: 2.9161x; 1.0663x over previous
import functools

import jax
import jax.numpy as jnp
from jax import lax
from jax.experimental import pallas as pl
from jax.experimental.pallas import tpu as pltpu
from jax.experimental.pallas import tpu_sc as plsc

VOCAB = 1000000
D = 64
HIDDEN = 84
OUT_D = 64
BATCH = 16384
HIST = 200

NC = 2
NS = 16
NW = NC * NS
S_PER_W = BATCH // NW
SB = 32
C = 40
T = HIST // C


def _pool_sc(x, emb):
    x1 = x.reshape(BATCH * HIST)
    mesh = plsc.VectorSubcoreMesh(core_axis_name="c", subcore_axis_name="s")

    @functools.partial(
        pl.kernel,
        mesh=mesh,
        compiler_params=pltpu.CompilerParams(use_tc_tiling_on_sc=False),
        out_type=jax.ShapeDtypeStruct((BATCH * D,), jnp.float32),
        scratch_types=[
            pltpu.VMEM((SB * HIST,), jnp.int32),
            pltpu.VMEM((C, D), jnp.float32),
            pltpu.VMEM((C, D), jnp.float32),
            pltpu.VMEM((SB * D,), jnp.float32),
            pltpu.SemaphoreType.DMA,
            pltpu.SemaphoreType.DMA,
        ],
    )
    def pool(x_hbm, emb_hbm, out_hbm, idx_v, acc0, acc1, ostage_v, sem0, sem1):
        wid = lax.axis_index("s") * NC + lax.axis_index("c")
        base = wid * S_PER_W
        inv = jnp.float32(1.0 / HIST)
        z = jnp.zeros((16,), jnp.float32)

        def zero(acc):
            def zbody(j, carry):
                for u in range(2):
                    acc[2 * j + u, pl.ds(0, 16)] = z
                    acc[2 * j + u, pl.ds(16, 16)] = z
                    acc[2 * j + u, pl.ds(32, 16)] = z
                    acc[2 * j + u, pl.ds(48, 16)] = z
                return carry
            lax.fori_loop(0, C // 2, zbody, 0)

        def fire(s, acc, sem):
            for t in range(T):
                pltpu.async_copy(
                    emb_hbm.at[idx_v.at[pl.ds(s * HIST + t * C, C)]],
                    acc, sem, add=True)

        def wait_all(acc, sem):
            cp = pltpu.make_async_copy(
                emb_hbm.at[idx_v.at[pl.ds(0, C)]], acc, sem)
            for t in range(T):
                cp.wait()

        def reduce(s, acc):
            def rbody(j, accs):
                a0, a1, a2, a3 = accs
                for u in range(2):
                    a0 = a0 + acc[2 * j + u, pl.ds(0, 16)]
                    a1 = a1 + acc[2 * j + u, pl.ds(16, 16)]
                    a2 = a2 + acc[2 * j + u, pl.ds(32, 16)]
                    a3 = a3 + acc[2 * j + u, pl.ds(48, 16)]
                return (a0, a1, a2, a3)
            a0, a1, a2, a3 = lax.fori_loop(0, C // 2, rbody, (z, z, z, z))
            ostage_v[pl.ds(s * D + 0, 16)] = a0 * inv
            ostage_v[pl.ds(s * D + 16, 16)] = a1 * inv
            ostage_v[pl.ds(s * D + 32, 16)] = a2 * inv
            ostage_v[pl.ds(s * D + 48, 16)] = a3 * inv

        def blk_body(blk, carry):
            sbase = base + blk * SB
            pltpu.sync_copy(x_hbm.at[pl.ds(sbase * HIST, SB * HIST)], idx_v)
            zero(acc0)
            fire(0, acc0, sem0)

            def pair_body(p, carry2):
                zero(acc1)
                fire(2 * p + 1, acc1, sem1)
                wait_all(acc0, sem0)
                reduce(2 * p, acc0)
                zero(acc0)

                @pl.when(p < SB // 2 - 1)
                def _():
                    fire(2 * p + 2, acc0, sem0)

                wait_all(acc1, sem1)
                reduce(2 * p + 1, acc1)
                return carry2

            lax.fori_loop(0, SB // 2, pair_body, 0)
            pltpu.sync_copy(ostage_v, out_hbm.at[pl.ds(sbase * D, SB * D)])
            return carry

        lax.fori_loop(0, S_PER_W // SB, blk_body, 0)

    return pool(x1, emb).reshape(BATCH, D)


def _mlp_body(p_ref, w1_ref, b1_ref, w2_ref, b2_ref, o_ref):
    h = jnp.tanh(
        jnp.dot(p_ref[...], w1_ref[...], preferred_element_type=jnp.float32)
        + b1_ref[...])
    o_ref[...] = jnp.tanh(
        jnp.dot(h, w2_ref[...], preferred_element_type=jnp.float32)
        + b2_ref[...])


def _mlp_tc(pooled, W1, b1, W2, b2):
    MB = 2048
    return pl.pallas_call(
        _mlp_body,
        grid=(BATCH // MB,),
        in_specs=[
            pl.BlockSpec((MB, D), lambda i: (i, 0)),
            pl.BlockSpec((D, HIDDEN), lambda i: (0, 0)),
            pl.BlockSpec((1, HIDDEN), lambda i: (0, 0)),
            pl.BlockSpec((HIDDEN, OUT_D), lambda i: (0, 0)),
            pl.BlockSpec((1, OUT_D), lambda i: (0, 0)),
        ],
        out_specs=pl.BlockSpec((MB, OUT_D), lambda i: (i, 0)),
        out_shape=jax.ShapeDtypeStruct((BATCH, OUT_D), jnp.float32),
    )(pooled, W1.T, b1[None, :], W2.T, b2[None, :])


def kernel(x, emb, W1, b1, W2, b2):
    pooled = _pool_sc(x, emb)
    return _mlp_tc(pooled, W1, b1, W2, b2)


# speedup vs baseline: 3.2344x; 1.1092x over previous
import functools

import jax
import jax.numpy as jnp
from jax import lax
from jax.experimental import pallas as pl
from jax.experimental.pallas import tpu as pltpu
from jax.experimental.pallas import tpu_sc as plsc

VOCAB = 1000000
D = 64
HIDDEN = 84
OUT_D = 64
BATCH = 16384
HIST = 200

NC = 2
NS = 16
NW = NC * NS
S_PER_W = BATCH // NW
SB = 64
C = 40
T = HIST // C


def _pool_sc(x, emb):
    x1 = x.reshape(BATCH * HIST)
    mesh = plsc.VectorSubcoreMesh(core_axis_name="c", subcore_axis_name="s")

    @functools.partial(
        pl.kernel,
        mesh=mesh,
        compiler_params=pltpu.CompilerParams(use_tc_tiling_on_sc=False),
        out_type=jax.ShapeDtypeStruct((BATCH * D,), jnp.float32),
        scratch_types=[
            pltpu.VMEM((SB * HIST,), jnp.int32),
            pltpu.VMEM((C, D), jnp.float32),
            pltpu.VMEM((C, D), jnp.float32),
            pltpu.VMEM((C, D), jnp.float32),
            pltpu.VMEM((C, D), jnp.float32),
            pltpu.VMEM((SB * D,), jnp.float32),
            pltpu.SemaphoreType.DMA,
            pltpu.SemaphoreType.DMA,
            pltpu.SemaphoreType.DMA,
            pltpu.SemaphoreType.DMA,
        ],
    )
    def pool(x_hbm, emb_hbm, out_hbm, idx_v, acc0, acc1, acc2, acc3,
             ostage_v, sem0, sem1, sem2, sem3):
        accs_sems = ((acc0, sem0), (acc1, sem1), (acc2, sem2), (acc3, sem3))
        wid = lax.axis_index("s") * NC + lax.axis_index("c")
        base = wid * S_PER_W
        inv = jnp.float32(1.0 / HIST)
        z = jnp.zeros((16,), jnp.float32)

        def zero(acc):
            def zbody(j, carry):
                for u in range(2):
                    acc[2 * j + u, pl.ds(0, 16)] = z
                    acc[2 * j + u, pl.ds(16, 16)] = z
                    acc[2 * j + u, pl.ds(32, 16)] = z
                    acc[2 * j + u, pl.ds(48, 16)] = z
                return carry
            lax.fori_loop(0, C // 2, zbody, 0)

        def fire(s, acc, sem):
            for t in range(T):
                pltpu.async_copy(
                    emb_hbm.at[idx_v.at[pl.ds(s * HIST + t * C, C)]],
                    acc, sem, add=True)

        def wait_all(acc, sem):
            cp = pltpu.make_async_copy(
                emb_hbm.at[idx_v.at[pl.ds(0, C)]], acc, sem)
            for t in range(T):
                cp.wait()

        def reduce_and_rezero(s, acc):
            def rbody(j, accs):
                a0, a1, a2, a3 = accs
                for u in range(2):
                    a0 = a0 + acc[2 * j + u, pl.ds(0, 16)]
                    a1 = a1 + acc[2 * j + u, pl.ds(16, 16)]
                    a2 = a2 + acc[2 * j + u, pl.ds(32, 16)]
                    a3 = a3 + acc[2 * j + u, pl.ds(48, 16)]
                    acc[2 * j + u, pl.ds(0, 16)] = z
                    acc[2 * j + u, pl.ds(16, 16)] = z
                    acc[2 * j + u, pl.ds(32, 16)] = z
                    acc[2 * j + u, pl.ds(48, 16)] = z
                return (a0, a1, a2, a3)
            a0, a1, a2, a3 = lax.fori_loop(0, C // 2, rbody, (z, z, z, z))
            ostage_v[pl.ds(s * D + 0, 16)] = a0 * inv
            ostage_v[pl.ds(s * D + 16, 16)] = a1 * inv
            ostage_v[pl.ds(s * D + 32, 16)] = a2 * inv
            ostage_v[pl.ds(s * D + 48, 16)] = a3 * inv

        for acc, _ in accs_sems:
            zero(acc)

        def blk_body(blk, carry):
            sbase = base + blk * SB
            pltpu.sync_copy(x_hbm.at[pl.ds(sbase * HIST, SB * HIST)], idx_v)
            for u, (acc, sem) in enumerate(accs_sems):
                fire(u, acc, sem)

            def quad_body(q, carry2):
                for u, (acc, sem) in enumerate(accs_sems):
                    wait_all(acc, sem)
                    reduce_and_rezero(4 * q + u, acc)

                    @pl.when(q < SB // 4 - 1)
                    def _():
                        fire(4 * q + 4 + u, acc, sem)

                return carry2

            lax.fori_loop(0, SB // 4, quad_body, 0)
            pltpu.sync_copy(ostage_v, out_hbm.at[pl.ds(sbase * D, SB * D)])
            return carry

        lax.fori_loop(0, S_PER_W // SB, blk_body, 0)

    return pool(x1, emb).reshape(BATCH, D)


def _mlp_body(p_ref, w1_ref, b1_ref, w2_ref, b2_ref, o_ref):
    h = jnp.tanh(
        jnp.dot(p_ref[...], w1_ref[...], preferred_element_type=jnp.float32)
        + b1_ref[...])
    o_ref[...] = jnp.tanh(
        jnp.dot(h, w2_ref[...], preferred_element_type=jnp.float32)
        + b2_ref[...])


def _mlp_tc(pooled, W1, b1, W2, b2):
    MB = 2048
    return pl.pallas_call(
        _mlp_body,
        grid=(BATCH // MB,),
        in_specs=[
            pl.BlockSpec((MB, D), lambda i: (i, 0)),
            pl.BlockSpec((D, HIDDEN), lambda i: (0, 0)),
            pl.BlockSpec((1, HIDDEN), lambda i: (0, 0)),
            pl.BlockSpec((HIDDEN, OUT_D), lambda i: (0, 0)),
            pl.BlockSpec((1, OUT_D), lambda i: (0, 0)),
        ],
        out_specs=pl.BlockSpec((MB, OUT_D), lambda i: (i, 0)),
        out_shape=jax.ShapeDtypeStruct((BATCH, OUT_D), jnp.float32),
    )(pooled, W1.T, b1[None, :], W2.T, b2[None, :])


def kernel(x, emb, W1, b1, W2, b2):
    pooled = _pool_sc(x, emb)
    return _mlp_tc(pooled, W1, b1, W2, b2)


# speedup vs baseline: 3.2768x; 1.0131x over previous
import functools

import jax
import jax.numpy as jnp
from jax import lax
from jax.experimental import pallas as pl
from jax.experimental.pallas import tpu as pltpu
from jax.experimental.pallas import tpu_sc as plsc

VOCAB = 1000000
D = 64
HIDDEN = 84
OUT_D = 64
BATCH = 16384
HIST = 200

NC = 2
NS = 16
NW = NC * NS
S_PER_W = BATCH // NW
SB = 64
C = 8
T = HIST // C


def _pool_sc(x, emb):
    x1 = x.reshape(BATCH * HIST)
    mesh = plsc.VectorSubcoreMesh(core_axis_name="c", subcore_axis_name="s")

    @functools.partial(
        pl.kernel,
        mesh=mesh,
        compiler_params=pltpu.CompilerParams(use_tc_tiling_on_sc=False),
        out_type=jax.ShapeDtypeStruct((BATCH * D,), jnp.float32),
        scratch_types=[
            pltpu.VMEM((SB * HIST,), jnp.int32),
            pltpu.VMEM((C, D), jnp.float32),
            pltpu.VMEM((C, D), jnp.float32),
            pltpu.VMEM((C, D), jnp.float32),
            pltpu.VMEM((C, D), jnp.float32),
            pltpu.VMEM((SB * D,), jnp.float32),
            pltpu.SemaphoreType.DMA,
            pltpu.SemaphoreType.DMA,
            pltpu.SemaphoreType.DMA,
            pltpu.SemaphoreType.DMA,
        ],
    )
    def pool(x_hbm, emb_hbm, out_hbm, idx_v, acc0, acc1, acc2, acc3,
             ostage_v, sem0, sem1, sem2, sem3):
        accs_sems = ((acc0, sem0), (acc1, sem1), (acc2, sem2), (acc3, sem3))
        wid = lax.axis_index("s") * NC + lax.axis_index("c")
        base = wid * S_PER_W
        inv = jnp.float32(1.0 / HIST)
        z = jnp.zeros((16,), jnp.float32)

        def zero(acc):
            def zbody(j, carry):
                for u in range(2):
                    acc[2 * j + u, pl.ds(0, 16)] = z
                    acc[2 * j + u, pl.ds(16, 16)] = z
                    acc[2 * j + u, pl.ds(32, 16)] = z
                    acc[2 * j + u, pl.ds(48, 16)] = z
                return carry
            lax.fori_loop(0, C // 2, zbody, 0)

        def fire(s, acc, sem):
            for t in range(T):
                pltpu.async_copy(
                    emb_hbm.at[idx_v.at[pl.ds(s * HIST + t * C, C)]],
                    acc, sem, add=True)

        def wait_all(acc, sem):
            cp = pltpu.make_async_copy(
                emb_hbm.at[idx_v.at[pl.ds(0, C)]], acc, sem)
            for t in range(T):
                cp.wait()

        def reduce_and_rezero(s, acc):
            def rbody(j, accs):
                a0, a1, a2, a3 = accs
                for u in range(2):
                    a0 = a0 + acc[2 * j + u, pl.ds(0, 16)]
                    a1 = a1 + acc[2 * j + u, pl.ds(16, 16)]
                    a2 = a2 + acc[2 * j + u, pl.ds(32, 16)]
                    a3 = a3 + acc[2 * j + u, pl.ds(48, 16)]
                    acc[2 * j + u, pl.ds(0, 16)] = z
                    acc[2 * j + u, pl.ds(16, 16)] = z
                    acc[2 * j + u, pl.ds(32, 16)] = z
                    acc[2 * j + u, pl.ds(48, 16)] = z
                return (a0, a1, a2, a3)
            a0, a1, a2, a3 = lax.fori_loop(0, C // 2, rbody, (z, z, z, z))
            ostage_v[pl.ds(s * D + 0, 16)] = a0 * inv
            ostage_v[pl.ds(s * D + 16, 16)] = a1 * inv
            ostage_v[pl.ds(s * D + 32, 16)] = a2 * inv
            ostage_v[pl.ds(s * D + 48, 16)] = a3 * inv

        for acc, _ in accs_sems:
            zero(acc)

        def blk_body(blk, carry):
            sbase = base + blk * SB
            pltpu.sync_copy(x_hbm.at[pl.ds(sbase * HIST, SB * HIST)], idx_v)
            for u, (acc, sem) in enumerate(accs_sems):
                fire(u, acc, sem)

            def quad_body(q, carry2):
                for u, (acc, sem) in enumerate(accs_sems):
                    wait_all(acc, sem)
                    reduce_and_rezero(4 * q + u, acc)

                    @pl.when(q < SB // 4 - 1)
                    def _():
                        fire(4 * q + 4 + u, acc, sem)

                return carry2

            lax.fori_loop(0, SB // 4, quad_body, 0)
            pltpu.sync_copy(ostage_v, out_hbm.at[pl.ds(sbase * D, SB * D)])
            return carry

        lax.fori_loop(0, S_PER_W // SB, blk_body, 0)

    return pool(x1, emb).reshape(BATCH, D)


def _mlp_body(p_ref, w1_ref, b1_ref, w2_ref, b2_ref, o_ref):
    h = jnp.tanh(
        jnp.dot(p_ref[...], w1_ref[...], preferred_element_type=jnp.float32)
        + b1_ref[...])
    o_ref[...] = jnp.tanh(
        jnp.dot(h, w2_ref[...], preferred_element_type=jnp.float32)
        + b2_ref[...])


def _mlp_tc(pooled, W1, b1, W2, b2):
    MB = 2048
    return pl.pallas_call(
        _mlp_body,
        grid=(BATCH // MB,),
        in_specs=[
            pl.BlockSpec((MB, D), lambda i: (i, 0)),
            pl.BlockSpec((D, HIDDEN), lambda i: (0, 0)),
            pl.BlockSpec((1, HIDDEN), lambda i: (0, 0)),
            pl.BlockSpec((HIDDEN, OUT_D), lambda i: (0, 0)),
            pl.BlockSpec((1, OUT_D), lambda i: (0, 0)),
        ],
        out_specs=pl.BlockSpec((MB, OUT_D), lambda i: (i, 0)),
        out_shape=jax.ShapeDtypeStruct((BATCH, OUT_D), jnp.float32),
    )(pooled, W1.T, b1[None, :], W2.T, b2[None, :])


def kernel(x, emb, W1, b1, W2, b2):
    pooled = _pool_sc(x, emb)
    return _mlp_tc(pooled, W1, b1, W2, b2)
